# baseline (speedup 1.0000x reference)
import functools

import jax
import jax.numpy as jnp
from jax import lax
from jax.experimental import pallas as pl
from jax.experimental.pallas import tpu as pltpu
from jax.experimental.pallas import tpu_sc as plsc

_REPACK_CH = 8192


def _repack_body(*refs):
    n = len(refs) // 2
    for in_ref, out_ref in zip(refs[:n], refs[n:]):
        x = in_ref[...]
        x3 = x.reshape(16, _REPACK_CH // 8, 8)
        y = jnp.transpose(x3, (1, 2, 0))
        out_ref[...] = y.reshape(_REPACK_CH // 8, 128)


def _tc_repack(*tables_t):
    n_rows = tables_t[0].shape[1]
    np_rows = n_rows // 8
    nblk = (n_rows + _REPACK_CH - 1) // _REPACK_CH
    n = len(tables_t)
    return pl.pallas_call(
        _repack_body,
        grid=(nblk,),
        in_specs=[pl.BlockSpec((16, _REPACK_CH), lambda i: (0, i))] * n,
        out_specs=[
            pl.BlockSpec((_REPACK_CH // 8, 128), lambda i: (i, 0))
        ] * n,
        out_shape=[jax.ShapeDtypeStruct((np_rows, 128), jnp.float32)] * n,
    )(*tables_t)


_CHG = 64


def _sc_gather_body(pu_hbm, pi_hbm, t0_hbm, t1_hbm, t2_hbm, t3_hbm,
                    o0_hbm, o1_hbm, o2_hbm, o3_hbm,
                    pu_v, pi_v, bufs, sem, *, nc):
    wid = lax.axis_index("s") * nc + lax.axis_index("c")
    n_chunk = pu_v.shape[0] * (pu_v.shape[1] // _CHG)
    bpw = n_chunk * _CHG
    base = wid * bpw
    pltpu.sync_copy(pu_hbm.at[wid], pu_v)
    pltpu.sync_copy(pi_hbm.at[wid], pi_v)
    tables = (t0_hbm, t1_hbm, t2_hbm, t3_hbm)
    outs = (o0_hbm, o1_hbm, o2_hbm, o3_hbm)
    per_row = pu_v.shape[1] // _CHG

    def idx_slice(iv, j):
        return iv.at[j // per_row, pl.ds((j % per_row) * _CHG, _CHG)]

    def fire(j):
        p = j % 2
        cps = []
        for t in range(4):
            iv = pu_v if t % 2 == 0 else pi_v
            cps.append(pltpu.async_copy(
                tables[t].at[idx_slice(iv, j)], bufs.at[t, p], sem))
        return cps

    pend = fire(0)
    for j in range(n_chunk):
        cur, pend = pend, (fire(j + 1) if j + 1 < n_chunk else [])
        for c in cur:
            c.wait()
        p = j % 2
        for t in range(4):
            pltpu.sync_copy(
                bufs.at[t, p], outs[t].at[pl.ds(base + j * _CHG, _CHG)])


def _sc_gather(pu3, pi3, p0, p1, p2, p3):
    info = plsc.get_sparse_core_info()
    nc, ns = info.num_cores, info.num_subcores
    nw = nc * ns
    b = pu3.shape[0] * pu3.shape[1] * pu3.shape[2] // nw * nw
    b = pu3.size
    f32 = jnp.float32
    run = pl.kernel(
        functools.partial(_sc_gather_body, nc=nc),
        mesh=plsc.VectorSubcoreMesh(core_axis_name="c", subcore_axis_name="s"),
        out_type=(
            jax.ShapeDtypeStruct((b, 128), f32),
            jax.ShapeDtypeStruct((b, 128), f32),
            jax.ShapeDtypeStruct((b, 128), f32),
            jax.ShapeDtypeStruct((b, 128), f32),
        ),
        scratch_types=[
            pltpu.VMEM(pu3.shape[1:], jnp.int32),
            pltpu.VMEM(pu3.shape[1:], jnp.int32),
            pltpu.VMEM((4, 2, _CHG, 128), f32),
            pltpu.SemaphoreType.DMA,
        ],
        compiler_params=pltpu.CompilerParams(use_tc_tiling_on_sc=True),
    )
    return run(pu3, pi3, p0, p1, p2, p3)


def _tc_dense_body(gumf_ref, gimf_ref, gumlp_ref, gimlp_ref, su_ref, si_ref,
                   w1u_ref, w1i_ref, b1_ref, w2_ref, b2_ref,
                   wpmf_ref, wph_ref, bp_ref, out_ref):
    f32 = jnp.float32
    bb = gumf_ref.shape[0]
    lane_grp = lax.broadcasted_iota(jnp.int32, (bb, 128), 1) // 16
    sel = (lax.broadcasted_iota(jnp.int32, (128, 16), 0) % 16
           == lax.broadcasted_iota(jnp.int32, (128, 16), 1)).astype(f32)
    mu = (lane_grp == su_ref[...]).astype(f32)
    mi = (lane_grp == si_ref[...]).astype(f32)

    def extract(g_ref, m):
        return jnp.dot(g_ref[...] * m, sel, preferred_element_type=f32)

    umf = extract(gumf_ref, mu)
    imf = extract(gimf_ref, mi)
    umlp = extract(gumlp_ref, mu)
    imlp = extract(gimlp_ref, mi)

    h1 = jnp.dot(umlp, w1u_ref[...], preferred_element_type=f32)
    h1 = h1 + jnp.dot(imlp, w1i_ref[...], preferred_element_type=f32)
    h1 = jnp.maximum(h1 + b1_ref[...], 0.0)
    h2 = jnp.dot(h1, w2_ref[...], preferred_element_type=f32) + b2_ref[...]
    h2 = jnp.maximum(h2, 0.0)
    mf = umf * imf
    z = (jnp.sum(mf * wpmf_ref[...], axis=1, keepdims=True)
         + jnp.sum(h2 * wph_ref[...], axis=1, keepdims=True)
         + bp_ref[0, 0])
    out_ref[...] = 1.0 / (1.0 + jnp.exp(-z))


def _tc_dense(gumf, gimf, gumlp, gimlp, su, si, W1, b1, W2, b2, Wp, bp):
    b = gumf.shape[0]
    dmf = 16
    dmlp = 16
    h1d = W1.shape[1]
    h2d = W2.shape[1]
    n_blocks = 8
    bb = b // n_blocks

    w1u = W1[:dmlp, :]
    w1i = W1[dmlp:, :]
    b1r = b1.reshape(1, h1d)
    b2r = b2.reshape(1, h2d)
    wpmf = Wp[:dmf, 0].reshape(1, dmf)
    wph = Wp[dmf:, 0].reshape(1, h2d)
    bpr = bp.reshape(1, 1)

    row = lambda i: (i, 0)
    fix = lambda i: (0, 0)
    return pl.pallas_call(
        _tc_dense_body,
        grid=(n_blocks,),
        in_specs=[
            pl.BlockSpec((bb, 128), row),
            pl.BlockSpec((bb, 128), row),
            pl.BlockSpec((bb, 128), row),
            pl.BlockSpec((bb, 128), row),
            pl.BlockSpec((bb, 1), row),
            pl.BlockSpec((bb, 1), row),
            pl.BlockSpec((dmlp, h1d), fix),
            pl.BlockSpec((dmlp, h1d), fix),
            pl.BlockSpec((1, h1d), fix),
            pl.BlockSpec((h1d, h2d), fix),
            pl.BlockSpec((1, h2d), fix),
            pl.BlockSpec((1, dmf), fix),
            pl.BlockSpec((1, h2d), fix),
            pl.BlockSpec((1, 1), fix),
        ],
        out_specs=pl.BlockSpec((bb, 1), row),
        out_shape=jax.ShapeDtypeStruct((b, 1), jnp.float32),
    )(gumf, gimf, gumlp, gimlp, su, si, w1u, w1i, b1r, W2, b2r, wpmf, wph, bpr)


def kernel(user, item, u_mf_table, i_mf_table, u_mlp_table, i_mlp_table,
           W1, b1, W2, b2, Wp, bp):
    info = plsc.get_sparse_core_info()
    nw = info.num_cores * info.num_subcores
    b = user.shape[0]
    bpw = b // nw

    user = user.astype(jnp.int32)
    item = item.astype(jnp.int32)
    pu3 = (user >> 3).reshape(nw, bpw // 128, 128)
    pi3 = (item >> 3).reshape(nw, bpw // 128, 128)
    su = (user & 7).reshape(b, 1)
    si = (item & 7).reshape(b, 1)

    p_umf, p_imf, p_umlp, p_imlp = _tc_repack(
        u_mf_table.T, i_mf_table.T, u_mlp_table.T, i_mlp_table.T)
    gumf, gimf, gumlp, gimlp = _sc_gather(pu3, pi3, p_umf, p_imf, p_umlp, p_imlp)
    return _tc_dense(gumf, gimf, gumlp, gimlp, su, si, W1, b1, W2, b2, Wp, bp)


# speedup vs baseline: 12.3340x; 12.3340x over previous
import functools

import jax
import jax.numpy as jnp
from jax import lax
from jax.experimental import pallas as pl
from jax.experimental.pallas import tpu as pltpu
from jax.experimental.pallas import tpu_sc as plsc

_REPACK_CH = 8192


def _repack_body(*refs):
    n = len(refs) // 2
    for in_ref, out_ref in zip(refs[:n], refs[n:]):
        x = in_ref[...]
        x3 = x.reshape(16, 8, _REPACK_CH // 8)
        x8 = jnp.transpose(x3, (1, 0, 2)).reshape(128, _REPACK_CH // 8)
        out_ref[...] = jnp.transpose(x8)


def _tc_repack(*tables_t):
    n_rows = tables_t[0].shape[1]
    nblk = (n_rows + _REPACK_CH - 1) // _REPACK_CH
    np_rows = nblk * (_REPACK_CH // 8)
    n = len(tables_t)
    return pl.pallas_call(
        _repack_body,
        grid=(nblk,),
        in_specs=[pl.BlockSpec((16, _REPACK_CH), lambda i: (0, i))] * n,
        out_specs=[
            pl.BlockSpec((_REPACK_CH // 8, 128), lambda i: (i, 0))
        ] * n,
        out_shape=[jax.ShapeDtypeStruct((np_rows, 128), jnp.float32)] * n,
    )(*tables_t)


_CHG = 64


def _sc_gather_body(pu_hbm, pi_hbm, t0_hbm, t1_hbm, t2_hbm, t3_hbm,
                    o0_hbm, o1_hbm, o2_hbm, o3_hbm,
                    pu_v, pi_v, bufs, sem, *, nc):
    wid = lax.axis_index("s") * nc + lax.axis_index("c")
    n_chunk = pu_v.shape[0] * (pu_v.shape[1] // _CHG)
    bpw = n_chunk * _CHG
    base = wid * bpw
    pltpu.sync_copy(pu_hbm.at[wid], pu_v)
    pltpu.sync_copy(pi_hbm.at[wid], pi_v)
    tables = (t0_hbm, t1_hbm, t2_hbm, t3_hbm)
    outs = (o0_hbm, o1_hbm, o2_hbm, o3_hbm)
    per_row = pu_v.shape[1] // _CHG

    def idx_slice(iv, j):
        return iv.at[j // per_row, pl.ds((j % per_row) * _CHG, _CHG)]

    def fire(j):
        p = j % 2
        cps = []
        for t in range(4):
            iv = pu_v if t % 2 == 0 else pi_v
            cps.append(pltpu.async_copy(
                tables[t].at[idx_slice(iv, j)], bufs.at[t, p], sem))
        return cps

    pend = fire(0)
    for j in range(n_chunk):
        cur, pend = pend, (fire(j + 1) if j + 1 < n_chunk else [])
        for c in cur:
            c.wait()
        p = j % 2
        for t in range(4):
            pltpu.sync_copy(
                bufs.at[t, p], outs[t].at[pl.ds(base + j * _CHG, _CHG)])


def _sc_gather(pu3, pi3, p0, p1, p2, p3):
    info = plsc.get_sparse_core_info()
    nc, ns = info.num_cores, info.num_subcores
    nw = nc * ns
    b = pu3.shape[0] * pu3.shape[1] * pu3.shape[2] // nw * nw
    b = pu3.size
    f32 = jnp.float32
    run = pl.kernel(
        functools.partial(_sc_gather_body, nc=nc),
        mesh=plsc.VectorSubcoreMesh(core_axis_name="c", subcore_axis_name="s"),
        out_type=(
            jax.ShapeDtypeStruct((b, 128), f32),
            jax.ShapeDtypeStruct((b, 128), f32),
            jax.ShapeDtypeStruct((b, 128), f32),
            jax.ShapeDtypeStruct((b, 128), f32),
        ),
        scratch_types=[
            pltpu.VMEM(pu3.shape[1:], jnp.int32),
            pltpu.VMEM(pu3.shape[1:], jnp.int32),
            pltpu.VMEM((4, 2, _CHG, 128), f32),
            pltpu.SemaphoreType.DMA,
        ],
        compiler_params=pltpu.CompilerParams(use_tc_tiling_on_sc=True),
    )
    return run(pu3, pi3, p0, p1, p2, p3)


def _tc_dense_body(gumf_ref, gimf_ref, gumlp_ref, gimlp_ref, su_ref, si_ref,
                   w1u_ref, w1i_ref, b1_ref, w2_ref, b2_ref,
                   wpmf_ref, wph_ref, bp_ref, out_ref):
    f32 = jnp.float32
    bb = gumf_ref.shape[0]
    lane_grp = lax.broadcasted_iota(jnp.int32, (bb, 128), 1) // 16
    sel = (lax.broadcasted_iota(jnp.int32, (128, 16), 0) % 16
           == lax.broadcasted_iota(jnp.int32, (128, 16), 1)).astype(f32)
    mu = (lane_grp == su_ref[...]).astype(f32)
    mi = (lane_grp == si_ref[...]).astype(f32)

    def extract(g_ref, m):
        return jnp.dot(g_ref[...] * m, sel, preferred_element_type=f32)

    umf = extract(gumf_ref, mu)
    imf = extract(gimf_ref, mi)
    umlp = extract(gumlp_ref, mu)
    imlp = extract(gimlp_ref, mi)

    h1 = jnp.dot(umlp, w1u_ref[...], preferred_element_type=f32)
    h1 = h1 + jnp.dot(imlp, w1i_ref[...], preferred_element_type=f32)
    h1 = jnp.maximum(h1 + b1_ref[...], 0.0)
    h2 = jnp.dot(h1, w2_ref[...], preferred_element_type=f32) + b2_ref[...]
    h2 = jnp.maximum(h2, 0.0)
    mf = umf * imf
    z = (jnp.sum(mf * wpmf_ref[...], axis=1, keepdims=True)
         + jnp.sum(h2 * wph_ref[...], axis=1, keepdims=True)
         + bp_ref[0, 0])
    out_ref[...] = 1.0 / (1.0 + jnp.exp(-z))


def _tc_dense(gumf, gimf, gumlp, gimlp, su, si, W1, b1, W2, b2, Wp, bp):
    b = gumf.shape[0]
    dmf = 16
    dmlp = 16
    h1d = W1.shape[1]
    h2d = W2.shape[1]
    n_blocks = 8
    bb = b // n_blocks

    w1u = W1[:dmlp, :]
    w1i = W1[dmlp:, :]
    b1r = b1.reshape(1, h1d)
    b2r = b2.reshape(1, h2d)
    wpmf = Wp[:dmf, 0].reshape(1, dmf)
    wph = Wp[dmf:, 0].reshape(1, h2d)
    bpr = bp.reshape(1, 1)

    row = lambda i: (i, 0)
    fix = lambda i: (0, 0)
    return pl.pallas_call(
        _tc_dense_body,
        grid=(n_blocks,),
        in_specs=[
            pl.BlockSpec((bb, 128), row),
            pl.BlockSpec((bb, 128), row),
            pl.BlockSpec((bb, 128), row),
            pl.BlockSpec((bb, 128), row),
            pl.BlockSpec((bb, 1), row),
            pl.BlockSpec((bb, 1), row),
            pl.BlockSpec((dmlp, h1d), fix),
            pl.BlockSpec((dmlp, h1d), fix),
            pl.BlockSpec((1, h1d), fix),
            pl.BlockSpec((h1d, h2d), fix),
            pl.BlockSpec((1, h2d), fix),
            pl.BlockSpec((1, dmf), fix),
            pl.BlockSpec((1, h2d), fix),
            pl.BlockSpec((1, 1), fix),
        ],
        out_specs=pl.BlockSpec((bb, 1), row),
        out_shape=jax.ShapeDtypeStruct((b, 1), jnp.float32),
    )(gumf, gimf, gumlp, gimlp, su, si, w1u, w1i, b1r, W2, b2r, wpmf, wph, bpr)


def kernel(user, item, u_mf_table, i_mf_table, u_mlp_table, i_mlp_table,
           W1, b1, W2, b2, Wp, bp):
    info = plsc.get_sparse_core_info()
    nw = info.num_cores * info.num_subcores
    b = user.shape[0]
    bpw = b // nw

    user = user.astype(jnp.int32)
    item = item.astype(jnp.int32)
    pu = (user >> 13) * 1024 + (user & 1023)
    pi = (item >> 13) * 1024 + (item & 1023)
    pu3 = pu.reshape(nw, bpw // 128, 128)
    pi3 = pi.reshape(nw, bpw // 128, 128)
    su = ((user >> 10) & 7).reshape(b, 1)
    si = ((item >> 10) & 7).reshape(b, 1)

    p_umf, p_imf, p_umlp, p_imlp = _tc_repack(
        u_mf_table.T, i_mf_table.T, u_mlp_table.T, i_mlp_table.T)
    gumf, gimf, gumlp, gimlp = _sc_gather(pu3, pi3, p_umf, p_imf, p_umlp, p_imlp)
    return _tc_dense(gumf, gimf, gumlp, gimlp, su, si, W1, b1, W2, b2, Wp, bp)
